# baseline (speedup 1.0000x reference)
import jax
import jax.numpy as jnp
from jax import lax
from jax.experimental import pallas as pl
from jax.experimental.pallas import tpu as pltpu
from jax.experimental.pallas import tpu_sc as plsc

_A = 8
_B = 16384
_T = 32
_NB = 64
_TABLE = _T * _NB * _NB
_HALF = _TABLE // 2
_HT = _T // 2
_NW = 32
_CHUNK = (_A * _B) // _NW
_LANES = 16


def _tile_q_body(state_hbm, w_hbm, out_hbm, tbl, st, acc):
    wid = lax.axis_index("s") * 2 + lax.axis_index("c")
    base = wid * _CHUNK
    act = wid // 4
    pltpu.sync_copy(state_hbm.at[pl.ds(2 * base, 2 * _CHUNK)], st)

    lanes = lax.iota(jnp.int32, _LANES)

    for h in (0, 1):
        pltpu.sync_copy(w_hbm.at[act, pl.ds(h * _HALF, _HALF)], tbl)

        def chunk_body(i, carry, h=h):
            o = i * _LANES
            rows2 = (lanes + o) * 2
            v0 = plsc.load_gather(st, [rows2])
            v1 = plsc.load_gather(st, [rows2 + 1])
            a0 = jnp.zeros((_LANES,), jnp.float32)
            a1 = jnp.zeros((_LANES,), jnp.float32)
            for tl in range(_HT):
                tg = h * _HT + tl
                off = jnp.float32(tg / 2048.0)
                i0 = jnp.minimum((v0 + off) * 64.0, 63.0).astype(jnp.int32)
                i1 = jnp.minimum((v1 + off) * 64.0, 63.0).astype(jnp.int32)
                flat = tl * (_NB * _NB) + i0 * _NB + i1
                g = plsc.load_gather(tbl, [flat])
                if tl % 2 == 0:
                    a0 = a0 + g
                else:
                    a1 = a1 + g
            a = a0 + a1
            if h == 0:
                acc[pl.ds(o, _LANES)] = a
            else:
                plsc.addupdate(acc.at[pl.ds(o, _LANES)], a)
            return carry

        lax.fori_loop(0, _CHUNK // _LANES, chunk_body, 0)

    pltpu.sync_copy(acc, out_hbm.at[pl.ds(base, _CHUNK)])


def kernel(state, weights):
    mesh = plsc.VectorSubcoreMesh(core_axis_name="c", subcore_axis_name="s")
    run = pl.kernel(
        _tile_q_body,
        out_type=jax.ShapeDtypeStruct((_A * _B,), jnp.float32),
        mesh=mesh,
        compiler_params=pltpu.CompilerParams(needs_layout_passes=False),
        scratch_types=[
            pltpu.VMEM((_HALF,), jnp.float32),
            pltpu.VMEM((2 * _CHUNK,), jnp.float32),
            pltpu.VMEM((_CHUNK,), jnp.float32),
        ],
    )
    return run(state.reshape(_A * _B * 2), weights)


# speedup vs baseline: 2.2963x; 2.2963x over previous
import jax
import jax.numpy as jnp
from jax import lax
from jax.experimental import pallas as pl
from jax.experimental.pallas import tpu as pltpu
from jax.experimental.pallas import tpu_sc as plsc

_A = 8
_B = 16384
_T = 32
_NB = 64
_TABLE = _T * _NB * _NB
_HALF = _TABLE // 2
_HT = _T // 2
_NW = 32
_CHUNK = (_A * _B) // _NW
_LANES = 16


def _tile_q_body(s0_hbm, s1_hbm, w_hbm, out_hbm, tbl, s0, s1, acc):
    wid = lax.axis_index("s") * 2 + lax.axis_index("c")
    base = wid * _CHUNK
    act = wid // 4
    pltpu.sync_copy(s0_hbm.at[pl.ds(base, _CHUNK)], s0)
    pltpu.sync_copy(s1_hbm.at[pl.ds(base, _CHUNK)], s1)

    for h in (0, 1):
        pltpu.sync_copy(w_hbm.at[act, pl.ds(h * _HALF, _HALF)], tbl)

        def chunk_body(i, carry, h=h):
            o = i * _LANES
            v0 = s0[pl.ds(o, _LANES)]
            v1 = s1[pl.ds(o, _LANES)]
            a0 = jnp.zeros((_LANES,), jnp.float32)
            a1 = jnp.zeros((_LANES,), jnp.float32)
            for tl in range(_HT):
                tg = h * _HT + tl
                off = jnp.float32(tg / 2048.0)
                i0 = jnp.minimum((v0 + off) * 64.0, 63.0).astype(jnp.int32)
                i1 = jnp.minimum((v1 + off) * 64.0, 63.0).astype(jnp.int32)
                flat = tl * (_NB * _NB) + i0 * _NB + i1
                g = plsc.load_gather(tbl, [flat])
                if tl % 2 == 0:
                    a0 = a0 + g
                else:
                    a1 = a1 + g
            a = a0 + a1
            if h == 0:
                acc[pl.ds(o, _LANES)] = a
            else:
                plsc.addupdate(acc.at[pl.ds(o, _LANES)], a)
            return carry

        lax.fori_loop(0, _CHUNK // _LANES, chunk_body, 0)

    pltpu.sync_copy(acc, out_hbm.at[pl.ds(base, _CHUNK)])


def kernel(state, weights):
    mesh = plsc.VectorSubcoreMesh(core_axis_name="c", subcore_axis_name="s")
    run = pl.kernel(
        _tile_q_body,
        out_type=jax.ShapeDtypeStruct((_A * _B,), jnp.float32),
        mesh=mesh,
        compiler_params=pltpu.CompilerParams(needs_layout_passes=False),
        scratch_types=[
            pltpu.VMEM((_HALF,), jnp.float32),
            pltpu.VMEM((_CHUNK,), jnp.float32),
            pltpu.VMEM((_CHUNK,), jnp.float32),
            pltpu.VMEM((_CHUNK,), jnp.float32),
        ],
    )
    s0 = state[:, :, 0].reshape(-1)
    s1 = state[:, :, 1].reshape(-1)
    return run(s0, s1, weights)


# speedup vs baseline: 2.5690x; 1.1188x over previous
import jax
import jax.numpy as jnp
from jax import lax
from jax.experimental import pallas as pl
from jax.experimental.pallas import tpu as pltpu
from jax.experimental.pallas import tpu_sc as plsc

_A = 8
_B = 16384
_T = 32
_NB = 64
_TABLE = _T * _NB * _NB
_NQ = 4
_QTR = _TABLE // _NQ
_QT = _T // _NQ
_NW = 32
_CHUNK = (_A * _B) // _NW
_LANES = 16


def _tile_q_body(s0_hbm, s1_hbm, w_hbm, out_hbm, tbl0, tbl1, s0, s1, acc,
                 sem0, sem1):
    wid = lax.axis_index("s") * 2 + lax.axis_index("c")
    base = wid * _CHUNK
    act = wid // 4
    bufs = (tbl0, tbl1)
    sems = (sem0, sem1)

    def start(q):
        return pltpu.async_copy(
            w_hbm.at[act, pl.ds(q * _QTR, _QTR)], bufs[q % 2], sems[q % 2])

    copies = {0: start(0), 1: start(1)}
    pltpu.sync_copy(s0_hbm.at[pl.ds(base, _CHUNK)], s0)
    pltpu.sync_copy(s1_hbm.at[pl.ds(base, _CHUNK)], s1)

    for q in range(_NQ):
        tbl = bufs[q % 2]
        copies[q].wait()

        @plsc.parallel_loop(0, _CHUNK // _LANES, unroll=2)
        def chunk_body(i, q=q, tbl=tbl):
            o = i * _LANES
            v0 = s0[pl.ds(o, _LANES)]
            v1 = s1[pl.ds(o, _LANES)]
            a0 = jnp.zeros((_LANES,), jnp.float32)
            a1 = jnp.zeros((_LANES,), jnp.float32)
            for tl in range(_QT):
                tg = q * _QT + tl
                off = jnp.float32(tg / 2048.0)
                i0 = jnp.minimum((v0 + off) * 64.0, 63.0).astype(jnp.int32)
                i1 = jnp.minimum((v1 + off) * 64.0, 63.0).astype(jnp.int32)
                flat = tl * (_NB * _NB) + i0 * _NB + i1
                g = plsc.load_gather(tbl, [flat])
                if tl % 2 == 0:
                    a0 = a0 + g
                else:
                    a1 = a1 + g
            a = a0 + a1
            if q == 0:
                acc[pl.ds(o, _LANES)] = a
            else:
                plsc.addupdate(acc.at[pl.ds(o, _LANES)], a)

        if q + 2 < _NQ:
            copies[q + 2] = start(q + 2)

    pltpu.sync_copy(acc, out_hbm.at[pl.ds(base, _CHUNK)])


def kernel(state, weights):
    mesh = plsc.VectorSubcoreMesh(core_axis_name="c", subcore_axis_name="s")
    run = pl.kernel(
        _tile_q_body,
        out_type=jax.ShapeDtypeStruct((_A * _B,), jnp.float32),
        mesh=mesh,
        compiler_params=pltpu.CompilerParams(needs_layout_passes=False),
        scratch_types=[
            pltpu.VMEM((_QTR,), jnp.float32),
            pltpu.VMEM((_QTR,), jnp.float32),
            pltpu.VMEM((_CHUNK,), jnp.float32),
            pltpu.VMEM((_CHUNK,), jnp.float32),
            pltpu.VMEM((_CHUNK,), jnp.float32),
            pltpu.SemaphoreType.DMA,
            pltpu.SemaphoreType.DMA,
        ],
    )
    s0 = state[:, :, 0].reshape(-1)
    s1 = state[:, :, 1].reshape(-1)
    return run(s0, s1, weights)


# speedup vs baseline: 2.8121x; 1.0946x over previous
import jax
import jax.numpy as jnp
from jax import lax
from jax.experimental import pallas as pl
from jax.experimental.pallas import tpu as pltpu
from jax.experimental.pallas import tpu_sc as plsc

_A = 8
_B = 16384
_T = 32
_NB = 64
_TABLE = _T * _NB * _NB
_NQ = 4
_QTR = _TABLE // _NQ
_QT = _T // _NQ
_NW = 32
_CHUNK = (_A * _B) // _NW
_LANES = 16


def _tile_q_body(s0_hbm, s1_hbm, w_hbm, out_hbm, tbl0, tbl1, s0, s1, acc,
                 sem0, sem1):
    wid = lax.axis_index("s") * 2 + lax.axis_index("c")
    base = wid * _CHUNK
    act = wid // 4
    bufs = (tbl0, tbl1)
    sems = (sem0, sem1)

    def start(q):
        return pltpu.async_copy(
            w_hbm.at[act, pl.ds(q * _QTR, _QTR)], bufs[q % 2], sems[q % 2])

    copies = {0: start(0), 1: start(1)}
    pltpu.sync_copy(s0_hbm.at[pl.ds(base, _CHUNK)], s0)
    pltpu.sync_copy(s1_hbm.at[pl.ds(base, _CHUNK)], s1)

    for q in range(_NQ):
        tbl = bufs[q % 2]
        copies[q].wait()

        @plsc.parallel_loop(0, _CHUNK // _LANES, unroll=2)
        def chunk_body(i, q=q, tbl=tbl):
            o = i * _LANES
            w0 = s0[pl.ds(o, _LANES)] * 64.0
            w1 = s1[pl.ds(o, _LANES)] * 64.0
            a0 = jnp.zeros((_LANES,), jnp.float32)
            a1 = jnp.zeros((_LANES,), jnp.float32)
            for tl in range(_QT):
                tg = q * _QT + tl
                off = jnp.float32(tg / 32.0)
                i0 = jnp.minimum(w0 + off, 63.0).astype(jnp.int32)
                i1 = jnp.minimum(w1 + off, 63.0).astype(jnp.int32)
                flat = i0 * _NB + i1
                g = plsc.load_gather(tbl.at[pl.ds(tl * (_NB * _NB), _NB * _NB)], [flat])
                if tl % 2 == 0:
                    a0 = a0 + g
                else:
                    a1 = a1 + g
            a = a0 + a1
            if q == 0:
                acc[pl.ds(o, _LANES)] = a
            else:
                plsc.addupdate(acc.at[pl.ds(o, _LANES)], a)

        if q + 2 < _NQ:
            copies[q + 2] = start(q + 2)

    pltpu.sync_copy(acc, out_hbm.at[pl.ds(base, _CHUNK)])


def kernel(state, weights):
    mesh = plsc.VectorSubcoreMesh(core_axis_name="c", subcore_axis_name="s")
    run = pl.kernel(
        _tile_q_body,
        out_type=jax.ShapeDtypeStruct((_A * _B,), jnp.float32),
        mesh=mesh,
        compiler_params=pltpu.CompilerParams(needs_layout_passes=False),
        scratch_types=[
            pltpu.VMEM((_QTR,), jnp.float32),
            pltpu.VMEM((_QTR,), jnp.float32),
            pltpu.VMEM((_CHUNK,), jnp.float32),
            pltpu.VMEM((_CHUNK,), jnp.float32),
            pltpu.VMEM((_CHUNK,), jnp.float32),
            pltpu.SemaphoreType.DMA,
            pltpu.SemaphoreType.DMA,
        ],
    )
    s0 = state[:, :, 0].reshape(-1)
    s1 = state[:, :, 1].reshape(-1)
    return run(s0, s1, weights)


# speedup vs baseline: 2.8863x; 1.0264x over previous
import jax
import jax.numpy as jnp
from jax import lax
from jax.experimental import pallas as pl
from jax.experimental.pallas import tpu as pltpu
from jax.experimental.pallas import tpu_sc as plsc

_A = 8
_B = 16384
_T = 32
_NB = 64
_TABLE = _T * _NB * _NB
_NQ = 4
_QTR = _TABLE // _NQ
_QT = _T // _NQ
_NW = 32
_CHUNK = (_A * _B) // _NW
_LANES = 16


def _tile_q_body(s0_hbm, s1_hbm, w_hbm, out_hbm, tbl0, tbl1, s0, s1, acc,
                 sem0, sem1):
    wid = lax.axis_index("s") * 2 + lax.axis_index("c")
    base = wid * _CHUNK
    act = wid // 4
    bufs = (tbl0, tbl1)
    sems = (sem0, sem1)

    def start(q):
        return pltpu.async_copy(
            w_hbm.at[act, pl.ds(q * _QTR, _QTR)], bufs[q % 2], sems[q % 2])

    copies = {0: start(0), 1: start(1)}
    pltpu.sync_copy(s0_hbm.at[pl.ds(base, _CHUNK)], s0)
    pltpu.sync_copy(s1_hbm.at[pl.ds(base, _CHUNK)], s1)

    for q in range(_NQ):
        tbl = bufs[q % 2]
        copies[q].wait()

        @plsc.parallel_loop(0, _CHUNK // _LANES, unroll=4)
        def chunk_body(i, q=q, tbl=tbl):
            o = i * _LANES
            w0 = s0[pl.ds(o, _LANES)] * 64.0
            w1 = s1[pl.ds(o, _LANES)] * 64.0
            a0 = jnp.zeros((_LANES,), jnp.float32)
            a1 = jnp.zeros((_LANES,), jnp.float32)
            for tl in range(_QT):
                tg = q * _QT + tl
                off = jnp.float32(tg / 32.0)
                i0 = jnp.minimum(w0 + off, 63.0).astype(jnp.int32)
                i1 = jnp.minimum(w1 + off, 63.0).astype(jnp.int32)
                flat = i0 * _NB + i1
                g = plsc.load_gather(tbl.at[pl.ds(tl * (_NB * _NB), _NB * _NB)], [flat])
                if tl % 2 == 0:
                    a0 = a0 + g
                else:
                    a1 = a1 + g
            a = a0 + a1
            if q == 0:
                acc[pl.ds(o, _LANES)] = a
            else:
                plsc.addupdate(acc.at[pl.ds(o, _LANES)], a)

        if q + 2 < _NQ:
            copies[q + 2] = start(q + 2)

    pltpu.sync_copy(acc, out_hbm.at[pl.ds(base, _CHUNK)])


def kernel(state, weights):
    mesh = plsc.VectorSubcoreMesh(core_axis_name="c", subcore_axis_name="s")
    run = pl.kernel(
        _tile_q_body,
        out_type=jax.ShapeDtypeStruct((_A * _B,), jnp.float32),
        mesh=mesh,
        compiler_params=pltpu.CompilerParams(needs_layout_passes=False),
        scratch_types=[
            pltpu.VMEM((_QTR,), jnp.float32),
            pltpu.VMEM((_QTR,), jnp.float32),
            pltpu.VMEM((_CHUNK,), jnp.float32),
            pltpu.VMEM((_CHUNK,), jnp.float32),
            pltpu.VMEM((_CHUNK,), jnp.float32),
            pltpu.SemaphoreType.DMA,
            pltpu.SemaphoreType.DMA,
        ],
    )
    s0 = state[:, :, 0].reshape(-1)
    s1 = state[:, :, 1].reshape(-1)
    return run(s0, s1, weights)
